# Initial kernel scaffold; baseline (speedup 1.0000x reference)
import jax
import jax.numpy as jnp
from jax.experimental import pallas as pl


def kernel(x, token_emb, pos_emb):
    raise NotImplementedError("write your pallas kernel here")


# speedup vs baseline: 1.3413x; 1.3413x over previous
import functools

import jax
import jax.numpy as jnp
from jax import lax
from jax.experimental import pallas as pl
from jax.experimental.pallas import tpu as pltpu
from jax.experimental.pallas import tpu_sc as plsc

D_MODEL = 768
LANES = 16
VPR = D_MODEL // LANES
NC = 2
NS = 16
NW = NC * NS
CHUNK = 64


@functools.partial(jax.jit, static_argnums=(3, 4))
def _embed(x_flat, token_emb, pos_emb, n_rows, seq_len):
    b_per_w = n_rows // NW
    n_chunks = b_per_w // CHUNK
    mesh = plsc.VectorSubcoreMesh(core_axis_name="c", subcore_axis_name="s")

    @functools.partial(
        pl.kernel,
        out_type=jax.ShapeDtypeStruct((n_rows, D_MODEL), jnp.float32),
        mesh=mesh,
        scratch_types=[
            pltpu.VMEM((b_per_w,), jnp.int32),
            pltpu.VMEM((CHUNK, D_MODEL), jnp.float32),
            pltpu.VMEM((CHUNK, D_MODEL), jnp.float32),
            pltpu.SemaphoreType.DMA,
        ],
    )
    def body(x_hbm, tok_hbm, pos_hbm, out_hbm, idx_v, tok_buf, pos_buf, sem):
        wid = lax.axis_index("s") * NC + lax.axis_index("c")
        base = wid * b_per_w
        pos_base = lax.rem(base, seq_len)
        pltpu.sync_copy(x_hbm.at[pl.ds(base, b_per_w)], idx_v)
        for c in range(n_chunks):
            row0 = c * CHUNK
            gather = pltpu.async_copy(
                tok_hbm.at[idx_v.at[pl.ds(row0, CHUNK)]], tok_buf, sem
            )
            pltpu.sync_copy(pos_hbm.at[pl.ds(pos_base + row0, CHUNK)], pos_buf)
            gather.wait()

            def add_row(r, _):
                for k in range(VPR):
                    sl = pl.ds(k * LANES, LANES)
                    plsc.addupdate(tok_buf.at[r, sl], pos_buf[r, sl])
                return ()

            lax.fori_loop(0, CHUNK, add_row, (), unroll=False)
            pltpu.sync_copy(tok_buf, out_hbm.at[pl.ds(base + row0, CHUNK)])

    return body(x_flat, token_emb, pos_emb)


def kernel(x, token_emb, pos_emb):
    batch, seq = x.shape
    x_flat = x.reshape(batch * seq).astype(jnp.int32)
    out = _embed(x_flat, token_emb, pos_emb, batch * seq, seq)
    return out.reshape(batch, seq, D_MODEL)
